# baseline (speedup 1.0000x reference)
import jax
import jax.numpy as jnp
from jax.experimental import pallas as pl
from jax.experimental.pallas import tpu as pltpu

N = 10000
E = 160000
D = 256
DOUT = 128
H = 8
C = 256

NPAD = 10240


def _mm_bias_body(x_ref, w_ref, b_ref, o_ref):
    o_ref[...] = jnp.dot(x_ref[...], w_ref[...],
                         preferred_element_type=jnp.float32) + b_ref[...]


def _matmul_bias(x, w, b, block_rows=1024):
    m, k = x.shape
    n = w.shape[1]
    grid = (m // block_rows,)
    return pl.pallas_call(
        _mm_bias_body,
        grid=grid,
        in_specs=[
            pl.BlockSpec((block_rows, k), lambda i: (i, 0)),
            pl.BlockSpec((k, n), lambda i: (0, 0)),
            pl.BlockSpec((n,), lambda i: (0,)),
        ],
        out_specs=pl.BlockSpec((block_rows, n), lambda i: (i, 0)),
        out_shape=jax.ShapeDtypeStruct((m, n), jnp.float32),
    )(x, w, b)


def kernel(x, edge_index, W1, b1, W2, b2):
    src = edge_index[0]
    dst = edge_index[1]
    xp = jnp.zeros((NPAD, D), jnp.float32).at[:N].set(x)
    h = _matmul_bias(xp, W1, b1)[:N]
    agg = jax.ops.segment_sum(h[src], dst, num_segments=N)
    deg = jax.ops.segment_sum(jnp.ones((E,), jnp.float32), dst, num_segments=N)
    h = (agg + h) / (deg[:, None] + 1.0)
    bits = (h[:, :H] > 0).astype(jnp.int32)
    ids = jnp.sum(bits * (2 ** jnp.arange(H, dtype=jnp.int32)), axis=1)
    csum = jax.ops.segment_sum(h, ids, num_segments=C)
    cnt = jax.ops.segment_sum(jnp.ones((N,), jnp.float32), ids, num_segments=C)
    cmean = csum / jnp.maximum(cnt, 1.0)[:, None]
    z = cmean @ W2 + b2
    h2 = z[ids]
    agg2 = jax.ops.segment_sum(h2[src], dst, num_segments=N)
    h2 = (agg2 + h2) / (deg[:, None] + 1.0)
    return h2


# speedup vs baseline: 2.4681x; 2.4681x over previous
import jax
import jax.numpy as jnp
from jax import lax
from jax.experimental import pallas as pl
from jax.experimental.pallas import tpu as pltpu
from jax.experimental.pallas import tpu_sc as plsc

N = 10000
E = 160000
D = 256
DOUT = 128
HB = 8
C = 256

NPAD = 10240
HALF = 5120
NTS = 16
SH_ROWS = 5136
DUMP = 5120
CSH = 384
ECH = 80
ER = 2048
EPAD = ER * ECH
ERT = 128
NPT = 320
ESL = 32
NSL = 64

_MESH = plsc.VectorSubcoreMesh(core_axis_name="c", subcore_axis_name="s")


def _sca_body(h_hbm, src_hbm, dst_hbm, zeros8_hbm, ones8_hbm, agg_hbm,
              deg_hbm, src_st, ldst_st, rows, onz_vm, agg_sh, deg_sh):
    c = lax.axis_index("c")
    s = lax.axis_index("s")
    base = c * HALF
    zv = jnp.zeros((16,), jnp.float32)

    def zrow(i, _):
        for j in range(16):
            rows[i, pl.ds(j * 16, 16)] = zv
        return 0
    lax.fori_loop(0, ECH, zrow, 0)
    for k in range(4):
        pltpu.sync_copy(rows, agg_sh.at[pl.ds(s * NPT + k * ECH, ECH)])
    pltpu.sync_copy(zeros8_hbm, deg_sh.at[pl.ds(s * NPT, NPT)])

    @pl.when(s == 0)
    def _():
        pltpu.sync_copy(rows.at[pl.ds(0, 16)], agg_sh.at[pl.ds(DUMP, 16)])
        pltpu.sync_copy(zeros8_hbm.at[pl.ds(0, 16)],
                        deg_sh.at[pl.ds(DUMP, 16)])
    pltpu.sync_copy(ones8_hbm.at[pl.ds(0, ECH)], onz_vm)

    pltpu.sync_copy(src_hbm.at[pl.ds(s * ERT, ERT)], src_st)
    pltpu.sync_copy(dst_hbm.at[pl.ds(s * ERT, ERT)], ldst_st)

    dump_row = DUMP + s
    def idx_body(i, _):
        for j in range(5):
            dv = ldst_st[i, pl.ds(j * 16, 16)]
            ld = dv - base
            ok = (ld >= 0) & (ld < HALF)
            ldst_st[i, pl.ds(j * 16, 16)] = jnp.where(ok, ld, dump_row)
        return 0
    lax.fori_loop(0, ERT, idx_body, 0)

    plsc.subcore_barrier()

    def gs_body(i, _):
        pltpu.sync_copy(h_hbm.at[src_st.at[i]], rows)
        pltpu.sync_copy(rows, agg_sh.at[ldst_st.at[i]], add=True)
        pltpu.sync_copy(onz_vm, deg_sh.at[ldst_st.at[i]], add=True)
        return 0
    lax.fori_loop(0, ERT, gs_body, 0)

    plsc.subcore_barrier()

    for k in range(4):
        pltpu.sync_copy(agg_sh.at[pl.ds(s * NPT + k * ECH, ECH)],
                        agg_hbm.at[pl.ds(base + s * NPT + k * ECH, ECH)])
    pltpu.sync_copy(deg_sh.at[pl.ds(s * NPT, NPT)],
                    deg_hbm.at[pl.ds(base + s * NPT, NPT)])


def _sc_a(h, src2, dst2, zeros8, ones8):
    return pl.kernel(
        _sca_body,
        out_type=[
            jax.ShapeDtypeStruct((NPAD, D), jnp.float32),
            jax.ShapeDtypeStruct((NPAD, 8), jnp.float32),
        ],
        mesh=_MESH,
        compiler_params=pltpu.CompilerParams(use_tc_tiling_on_sc=False),
        scratch_types=[
            pltpu.VMEM((ERT, ECH), jnp.int32),
            pltpu.VMEM((ERT, ECH), jnp.int32),
            pltpu.VMEM((ECH, D), jnp.float32),
            pltpu.VMEM((ECH, 8), jnp.float32),
            pltpu.VMEM_SHARED((SH_ROWS, D), jnp.float32),
            pltpu.VMEM_SHARED((SH_ROWS, 8), jnp.float32),
        ],
    )(h, src2, dst2, zeros8, ones8)


def _scb_body(hp_hbm, ids_hbm, src_hbm, dst_hbm, csum_hbm, cnts_hbm,
              ids_vm, cflat, rows2, cid_st, se_st, de_st, csum_sh):
    c = lax.axis_index("c")
    s = lax.axis_index("s")
    gt = c * NTS + s
    nb = gt * NPT
    zv = jnp.zeros((16,), jnp.float32)
    ov = jnp.ones((16,), jnp.float32)
    iot = lax.iota(jnp.int32, 16)

    pltpu.sync_copy(ids_hbm, ids_vm)

    def zrow(i, _):
        for j in range(16):
            rows2[i, pl.ds(j * 16, 16)] = zv
        return 0
    lax.fori_loop(0, 64, zrow, 0)
    pltpu.sync_copy(rows2.at[pl.ds(0, 24)], csum_sh.at[pl.ds(s * 24, 24)])

    for i in range(20):
        iv = ids_vm[pl.ds(nb + i * 16, 16)]
        ok = (nb + i * 16 + iot) < N
        cid_st[i // 4, pl.ds((i % 4) * 16, 16)] = jnp.where(ok, iv, C)

    plsc.subcore_barrier()

    for j in range(5):
        pltpu.sync_copy(hp_hbm.at[pl.ds(nb + j * 64, 64)], rows2)
        pltpu.sync_copy(rows2, csum_sh.at[cid_st.at[j]], add=True)

    def zc(i, _):
        cflat[pl.ds(i * 16, 16)] = zv
        return 0
    lax.fori_loop(0, NPT * C // 16, zc, 0)

    def slab_body(t, _):
        pltpu.sync_copy(src_hbm.at[pl.ds(t * ESL, ESL)], se_st)
        pltpu.sync_copy(dst_hbm.at[pl.ds(t * ESL, ESL)], de_st)

        def row_body(r, _):
            for j in range(5):
                sv = se_st[r, pl.ds(j * 16, 16)]
                dv = de_st[r, pl.ds(j * 16, 16)]
                cid = plsc.load_gather(ids_vm, [sv])
                ld = dv - nb
                ok = (ld >= 0) & (ld < NPT)
                flat = jnp.where(ok, ld, 0) * C + cid
                plsc.addupdate_scatter(cflat, [flat], ov, mask=ok)
            return 0
        lax.fori_loop(0, ESL, row_body, 0)
        return 0
    lax.fori_loop(0, NSL, slab_body, 0)

    for i in range(20):
        iv = ids_vm[pl.ds(nb + i * 16, 16)]
        ok = (nb + i * 16 + iot) < N
        flat = (i * 16 + iot) * C + iv
        plsc.addupdate_scatter(cflat, [flat], ov, mask=ok)

    plsc.subcore_barrier()
    pltpu.sync_copy(csum_sh.at[pl.ds(s * 24, 24)],
                    csum_hbm.at[c, pl.ds(s * 24, 24)])
    pltpu.sync_copy(cflat, cnts_hbm.at[pl.ds(gt * (NPT * C), NPT * C)])


def _sc_b(hp, ids, src2, dst2):
    return pl.kernel(
        _scb_body,
        out_type=[
            jax.ShapeDtypeStruct((2, CSH, D), jnp.float32),
            jax.ShapeDtypeStruct((NPAD * C,), jnp.float32),
        ],
        mesh=_MESH,
        compiler_params=pltpu.CompilerParams(use_tc_tiling_on_sc=False,
                                             needs_layout_passes=False),
        scratch_types=[
            pltpu.VMEM((NPAD,), jnp.int32),
            pltpu.VMEM((NPT * C,), jnp.float32),
            pltpu.VMEM((64, D), jnp.float32),
            pltpu.VMEM((5, 64), jnp.int32),
            pltpu.VMEM((ESL, ECH), jnp.int32),
            pltpu.VMEM((ESL, ECH), jnp.int32),
            pltpu.VMEM_SHARED((CSH, D), jnp.float32),
        ],
    )(hp, ids, src2, dst2)


def _tc1_body(x_ref, w_ref, b_ref, o_ref):
    o_ref[...] = jnp.dot(x_ref[...], w_ref[...],
                         preferred_element_type=jnp.float32) + b_ref[...]


def _tc1(xp, W1, b1, block_rows=1024):
    grid = (NPAD // block_rows,)
    return pl.pallas_call(
        _tc1_body,
        grid=grid,
        in_specs=[
            pl.BlockSpec((block_rows, D), lambda i: (i, 0)),
            pl.BlockSpec((D, D), lambda i: (0, 0)),
            pl.BlockSpec((D,), lambda i: (0,)),
        ],
        out_specs=pl.BlockSpec((block_rows, D), lambda i: (i, 0)),
        out_shape=jax.ShapeDtypeStruct((NPAD, D), jnp.float32),
    )(xp, W1, b1)


def _tc2_body(agg_ref, h_ref, deg_ref, wsel_ref, hp_ref, ids_ref, cnt_ref):
    i = pl.program_id(0)
    deg = deg_ref[...]
    hp = (agg_ref[...] + h_ref[...]) / (deg[:, None] + 1.0)
    hp_ref[...] = hp
    bits = (hp > 0).astype(jnp.float32)
    idsf = jnp.dot(bits, wsel_ref[...], preferred_element_type=jnp.float32)
    ids = idsf[:, 0].astype(jnp.int32)
    ids_ref[...] = ids
    rows = hp.shape[0]
    gidx = i * rows + lax.broadcasted_iota(jnp.int32, (rows, 1), 0)
    onehot = ((ids[:, None] == lax.broadcasted_iota(jnp.int32, (rows, C), 1))
              & (gidx < N)).astype(jnp.float32)
    part = jnp.sum(onehot, axis=0)

    @pl.when(i == 0)
    def _():
        cnt_ref[...] = jnp.zeros_like(cnt_ref)
    cnt_ref[...] += part


def _tc2(agg, h, deg, wsel, block_rows=1024):
    grid = (NPAD // block_rows,)
    return pl.pallas_call(
        _tc2_body,
        grid=grid,
        in_specs=[
            pl.BlockSpec((block_rows, D), lambda i: (i, 0)),
            pl.BlockSpec((block_rows, D), lambda i: (i, 0)),
            pl.BlockSpec((block_rows,), lambda i: (i,)),
            pl.BlockSpec((D, DOUT), lambda i: (0, 0)),
        ],
        out_specs=[
            pl.BlockSpec((block_rows, D), lambda i: (i, 0)),
            pl.BlockSpec((block_rows,), lambda i: (i,)),
            pl.BlockSpec((C,), lambda i: (0,)),
        ],
        out_shape=[
            jax.ShapeDtypeStruct((NPAD, D), jnp.float32),
            jax.ShapeDtypeStruct((NPAD,), jnp.int32),
            jax.ShapeDtypeStruct((C,), jnp.float32),
        ],
    )(agg, h, deg, wsel)


def _tc3_body(counts_ref, deg_ref, csum_ref, cnt_ref, w2_ref, b2_ref, o_ref):
    csum = csum_ref[0, :C, :] + csum_ref[1, :C, :]
    cnt = cnt_ref[...]
    cmean = csum / jnp.maximum(cnt, 1.0)[:, None]
    z = jnp.dot(cmean, w2_ref[...], preferred_element_type=jnp.float32) \
        + b2_ref[...]
    agg2 = jnp.dot(counts_ref[...], z, preferred_element_type=jnp.float32)
    o_ref[...] = agg2 / (deg_ref[...][:, None] + 1.0)


def _tc3(counts, deg, csum, cnt, W2, b2, block_rows=1024):
    grid = (NPAD // block_rows,)
    return pl.pallas_call(
        _tc3_body,
        grid=grid,
        in_specs=[
            pl.BlockSpec((block_rows, C), lambda i: (i, 0)),
            pl.BlockSpec((block_rows,), lambda i: (i,)),
            pl.BlockSpec((2, CSH, D), lambda i: (0, 0, 0)),
            pl.BlockSpec((C,), lambda i: (0,)),
            pl.BlockSpec((D, DOUT), lambda i: (0, 0)),
            pl.BlockSpec((DOUT,), lambda i: (0,)),
        ],
        out_specs=pl.BlockSpec((block_rows, DOUT), lambda i: (i, 0)),
        out_shape=jax.ShapeDtypeStruct((NPAD, DOUT), jnp.float32),
    )(counts, deg, csum, cnt, W2, b2)


def kernel(x, edge_index, W1, b1, W2, b2):
    xp = jnp.zeros((NPAD, D), jnp.float32).at[:N].set(x)
    src2 = jnp.concatenate(
        [edge_index[0], jnp.zeros((EPAD - E,), jnp.int32)]).reshape(ER, ECH)
    dst2 = jnp.concatenate(
        [edge_index[1], jnp.full((EPAD - E,), NPAD, jnp.int32)]).reshape(ER, ECH)
    wsel = jnp.zeros((D, DOUT), jnp.float32).at[:HB, 0].set(
        (2 ** jnp.arange(HB)).astype(jnp.float32))

    zeros8 = jnp.zeros((NPT, 8), jnp.float32)
    ones8 = jnp.ones((NPT, 8), jnp.float32)

    h = _tc1(xp, W1, b1)
    agg, deg8 = _sc_a(h, src2, dst2, zeros8, ones8)
    deg = deg8[:, 0]
    hp, ids, cnt = _tc2(agg, h, deg, wsel)
    csum, cntsf = _sc_b(hp, ids, src2, dst2)
    counts = cntsf.reshape(NPAD, C)
    out = _tc3(counts, deg, csum, cnt, W2, b2)
    return out[:N]
